# baseline (speedup 1.0000x reference)
import functools

import jax
import jax.numpy as jnp
from jax import lax
from jax.experimental import pallas as pl
from jax.experimental.pallas import tpu as pltpu

_BB = 8


def _body(months_ref, ch_ref, pos_ref, mtab_ref, tok_ref, out_ref):
    tok = tok_ref[...]
    bb, t = tok.shape[0], tok.shape[1]
    n = tok.shape[3] // 4
    m = months_ref[0]
    mo = jnp.zeros((bb, t, n), jnp.float32)
    for k in range(12):
        sel = (m == k).astype(jnp.float32)[..., None]
        mo = mo + sel * mtab_ref[k, :][None, None, :]
    ch = ch_ref[...]
    pos = pos_ref[...]
    out_ref[..., 0:n] = tok[..., 0:n] + ch[None, None, :, :]
    out_ref[..., n:2 * n] = tok[..., n:2 * n] + pos[None, :, None, :]
    out_ref[..., 2 * n:3 * n] = tok[..., 2 * n:3 * n] + mo[:, :, None, :]
    out_ref[..., 3 * n:] = tok[..., 3 * n:]


@jax.jit
def kernel(modality_tokens, timestamps, channel_embed, pos_embed, month_tab):
    b, t, bs, d = modality_tokens.shape
    months = timestamps[:, :, 1].astype(jnp.int32).reshape(b // _BB, _BB, t)
    grid = (b // _BB,)
    return pl.pallas_call(
        _body,
        grid=grid,
        in_specs=[
            pl.BlockSpec((1, _BB, t), lambda i: (i, 0, 0)),
            pl.BlockSpec((bs, d // 4), lambda i: (0, 0)),
            pl.BlockSpec((t, d // 4), lambda i: (0, 0)),
            pl.BlockSpec((12, d // 4), lambda i: (0, 0)),
            pl.BlockSpec((_BB, t, bs, d), lambda i: (i, 0, 0, 0)),
        ],
        out_specs=pl.BlockSpec((_BB, t, bs, d), lambda i: (i, 0, 0, 0)),
        out_shape=jax.ShapeDtypeStruct((b, t, bs, d), jnp.float32),
        compiler_params=pltpu.CompilerParams(
            dimension_semantics=("arbitrary",),
        ),
    )(months, channel_embed, pos_embed[:t], month_tab, modality_tokens)
